# baseline (speedup 1.0000x reference)
import functools

import jax
import jax.numpy as jnp
from jax.experimental import pallas as pl
from jax.experimental.pallas import tpu as pltpu

_Q = 1024
_D = 384
_M = 50000
_BM = 1024
_QC = 256
_K = 5
_IPAD = 2.0 ** 30


def _normalize_bf16(x):
    n = jnp.sqrt(jnp.sum(x * x, axis=1, keepdims=True))
    return (x / jnp.clip(n, 1e-12)).astype(jnp.bfloat16)


def _dot_nt(a, b):
    return jax.lax.dot_general(
        a, b, (((1,), (1,)), ((), ())),
        preferred_element_type=jnp.float32)


def _topk_kernel(td, ti, to, md, mi, mo,
                 vals_out, idx_out,
                 tdn, tin, ton, rv, ri):
    m_step = pl.program_id(0)
    nm = pl.num_programs(0)

    @pl.when(m_step == 0)
    def _init():
        tdn[...] = _normalize_bf16(td[...])
        tin[...] = _normalize_bf16(ti[...])
        ton[...] = _normalize_bf16(to[...])
        rv[...] = jnp.full((_Q, 128), -jnp.inf, jnp.float32)
        ri[...] = jnp.full((_Q, 128), _IPAD, jnp.float32)

    mdn = _normalize_bf16(md[...])
    min_ = _normalize_bf16(mi[...])
    mon = _normalize_bf16(mo[...])

    lane = jax.lax.broadcasted_iota(jnp.int32, (_QC, _BM), 1).astype(jnp.float32)
    gcol = lane + (m_step * _BM)
    valid = gcol < float(_M)

    for qi in range(_Q // _QC):
        sl = slice(qi * _QC, (qi + 1) * _QC)
        s = (6.0 * _dot_nt(tdn[sl, :], mdn)
             + 2.0 * _dot_nt(tin[sl, :], min_)
             + 2.0 * _dot_nt(ton[sl, :], mon)) / 3.0
        s = jnp.where(valid, s, -jnp.inf)

        sx = jnp.concatenate([s, rv[sl, :]], axis=1)
        gx = jnp.concatenate([gcol, ri[sl, :]], axis=1)

        lane128 = jax.lax.broadcasted_iota(jnp.int32, (_QC, 128), 1)
        nvt = jnp.full((_QC, 128), -jnp.inf, jnp.float32)
        nit = jnp.full((_QC, 128), _IPAD, jnp.float32)
        for k in range(_K):
            mval = jnp.max(sx, axis=1, keepdims=True)
            midx = jnp.min(jnp.where(sx == mval, gx, jnp.inf),
                           axis=1, keepdims=True)
            nvt = jnp.where(lane128 == k, mval, nvt)
            nit = jnp.where(lane128 == k, midx, nit)
            sx = jnp.where(gx == midx, -jnp.inf, sx)
        rv[sl, :] = nvt
        ri[sl, :] = nit

    @pl.when(m_step == nm - 1)
    def _emit():
        vals_out[...] = rv[...][:, 0:_K]
        idx_out[...] = ri[...][:, 0:_K].astype(jnp.int32)


@jax.jit
def _run(task_desc, task_in, task_out, model_desc, model_in, model_out):
    nm = pl.cdiv(_M, _BM)
    q_spec = pl.BlockSpec((_Q, _D), lambda m: (0, 0))
    m_spec = pl.BlockSpec((_BM, _D), lambda m: (m, 0))
    out_spec = pl.BlockSpec((_Q, _K), lambda m: (0, 0))
    return pl.pallas_call(
        _topk_kernel,
        grid=(nm,),
        in_specs=[q_spec, q_spec, q_spec, m_spec, m_spec, m_spec],
        out_specs=[out_spec, out_spec],
        out_shape=[
            jax.ShapeDtypeStruct((_Q, _K), jnp.float32),
            jax.ShapeDtypeStruct((_Q, _K), jnp.int32),
        ],
        scratch_shapes=[
            pltpu.VMEM((_Q, _D), jnp.bfloat16),
            pltpu.VMEM((_Q, _D), jnp.bfloat16),
            pltpu.VMEM((_Q, _D), jnp.bfloat16),
            pltpu.VMEM((_Q, 128), jnp.float32),
            pltpu.VMEM((_Q, 128), jnp.float32),
        ],
    )(task_desc, task_in, task_out, model_desc, model_in, model_out)


def kernel(task_desc, task_in, task_out, model_desc, model_in, model_out, top_k):
    vals, idx = _run(task_desc, task_in, task_out,
                     model_desc, model_in, model_out)
    return vals, idx


# speedup vs baseline: 1.0825x; 1.0825x over previous
import functools

import jax
import jax.numpy as jnp
from jax.experimental import pallas as pl
from jax.experimental.pallas import tpu as pltpu

_Q = 1024
_D = 384
_M = 50000
_BM = 1024
_QC = 256
_K = 5
_IPAD = 2.0 ** 30


def _normalize_bf16(x):
    n = jnp.sqrt(jnp.sum(x * x, axis=1, keepdims=True))
    return (x / jnp.clip(n, 1e-12)).astype(jnp.bfloat16)


def _dot_nt(a, b):
    return jax.lax.dot_general(
        a, b, (((1,), (1,)), ((), ())),
        preferred_element_type=jnp.float32)


def _topk_kernel(td, ti, to, md, mi, mo,
                 vals_out, idx_out,
                 tdn, tio, rv, ri):
    m_step = pl.program_id(0)
    nm = pl.num_programs(0)

    @pl.when(m_step == 0)
    def _init():
        tdn[...] = _normalize_bf16(td[...])
        tio[:, 0:_D] = _normalize_bf16(ti[...])
        tio[:, _D:2 * _D] = _normalize_bf16(to[...])
        rv[...] = jnp.full((_Q, 128), -jnp.inf, jnp.float32)
        ri[...] = jnp.full((_Q, 128), _IPAD, jnp.float32)

    mdn = _normalize_bf16(md[...])
    mio = jnp.concatenate([_normalize_bf16(mi[...]),
                           _normalize_bf16(mo[...])], axis=1)

    lane = jax.lax.broadcasted_iota(jnp.int32, (_QC, _BM), 1).astype(jnp.float32)
    gcol = lane + (m_step * _BM)
    amask = jnp.where(gcol < float(_M), 0.0, -jnp.inf)
    w23 = jnp.float32(2.0) / jnp.float32(3.0)

    for qi in range(_Q // _QC):
        sl = slice(qi * _QC, (qi + 1) * _QC)
        s = (2.0 * _dot_nt(tdn[sl, :], mdn)
             + w23 * _dot_nt(tio[sl, :], mio)) + amask

        sx = jnp.concatenate([s, rv[sl, :]], axis=1)
        gx = jnp.concatenate([gcol, ri[sl, :]], axis=1)

        lane128 = jax.lax.broadcasted_iota(jnp.int32, (_QC, 128), 1)
        nvt = jnp.full((_QC, 128), -jnp.inf, jnp.float32)
        nit = jnp.full((_QC, 128), _IPAD, jnp.float32)
        for k in range(_K):
            mval = jnp.max(sx, axis=1, keepdims=True)
            midx = jnp.min(jnp.where(sx == mval, gx, jnp.inf),
                           axis=1, keepdims=True)
            nvt = jnp.where(lane128 == k, mval, nvt)
            nit = jnp.where(lane128 == k, midx, nit)
            sx = jnp.where(gx == midx, -jnp.inf, sx)
        rv[sl, :] = nvt
        ri[sl, :] = nit

    @pl.when(m_step == nm - 1)
    def _emit():
        vals_out[...] = rv[...][:, 0:_K]
        idx_out[...] = ri[...][:, 0:_K].astype(jnp.int32)


@jax.jit
def _run(task_desc, task_in, task_out, model_desc, model_in, model_out):
    nm = pl.cdiv(_M, _BM)
    q_spec = pl.BlockSpec((_Q, _D), lambda m: (0, 0))
    m_spec = pl.BlockSpec((_BM, _D), lambda m: (m, 0))
    out_spec = pl.BlockSpec((_Q, _K), lambda m: (0, 0))
    return pl.pallas_call(
        _topk_kernel,
        grid=(nm,),
        in_specs=[q_spec, q_spec, q_spec, m_spec, m_spec, m_spec],
        out_specs=[out_spec, out_spec],
        out_shape=[
            jax.ShapeDtypeStruct((_Q, _K), jnp.float32),
            jax.ShapeDtypeStruct((_Q, _K), jnp.int32),
        ],
        scratch_shapes=[
            pltpu.VMEM((_Q, _D), jnp.bfloat16),
            pltpu.VMEM((_Q, 2 * _D), jnp.bfloat16),
            pltpu.VMEM((_Q, 128), jnp.float32),
            pltpu.VMEM((_Q, 128), jnp.float32),
        ],
    )(task_desc, task_in, task_out, model_desc, model_in, model_out)


def kernel(task_desc, task_in, task_out, model_desc, model_in, model_out, top_k):
    vals, idx = _run(task_desc, task_in, task_out,
                     model_desc, model_in, model_out)
    return vals, idx


# speedup vs baseline: 1.0977x; 1.0140x over previous
import functools

import jax
import jax.numpy as jnp
from jax.experimental import pallas as pl
from jax.experimental.pallas import tpu as pltpu

_Q = 1024
_D = 384
_M = 50000
_BM = 2048
_QC = 256
_K = 5
_IPAD = 2.0 ** 30


def _normalize_bf16(x):
    n = jnp.sqrt(jnp.sum(x * x, axis=1, keepdims=True))
    return (x / jnp.clip(n, 1e-12)).astype(jnp.bfloat16)


def _dot_nt(a, b):
    return jax.lax.dot_general(
        a, b, (((1,), (1,)), ((), ())),
        preferred_element_type=jnp.float32)


def _topk_kernel(td, ti, to, md, mi, mo,
                 vals_out, idx_out,
                 tdn, tio, rv, ri):
    m_step = pl.program_id(0)
    nm = pl.num_programs(0)

    @pl.when(m_step == 0)
    def _init():
        tdn[...] = _normalize_bf16(td[...])
        tio[:, 0:_D] = _normalize_bf16(ti[...])
        tio[:, _D:2 * _D] = _normalize_bf16(to[...])
        rv[...] = jnp.full((_Q, 128), -jnp.inf, jnp.float32)
        ri[...] = jnp.full((_Q, 128), _IPAD, jnp.float32)

    mdn = _normalize_bf16(md[...])
    mio = jnp.concatenate([_normalize_bf16(mi[...]),
                           _normalize_bf16(mo[...])], axis=1)

    lane = jax.lax.broadcasted_iota(jnp.int32, (_QC, _BM), 1).astype(jnp.float32)
    gcol = lane + (m_step * _BM)
    amask = jnp.where(gcol < float(_M), 0.0, -jnp.inf)
    w23 = jnp.float32(2.0) / jnp.float32(3.0)

    for qi in range(_Q // _QC):
        sl = slice(qi * _QC, (qi + 1) * _QC)
        s = (2.0 * _dot_nt(tdn[sl, :], mdn)
             + w23 * _dot_nt(tio[sl, :], mio)) + amask

        sx = jnp.concatenate([s, rv[sl, :]], axis=1)
        gx = jnp.concatenate([gcol, ri[sl, :]], axis=1)

        lane128 = jax.lax.broadcasted_iota(jnp.int32, (_QC, 128), 1)
        nvt = jnp.full((_QC, 128), -jnp.inf, jnp.float32)
        nit = jnp.full((_QC, 128), _IPAD, jnp.float32)
        for k in range(_K):
            mval = jnp.max(sx, axis=1, keepdims=True)
            midx = jnp.min(jnp.where(sx == mval, gx, jnp.inf),
                           axis=1, keepdims=True)
            nvt = jnp.where(lane128 == k, mval, nvt)
            nit = jnp.where(lane128 == k, midx, nit)
            sx = jnp.where(gx == midx, -jnp.inf, sx)
        rv[sl, :] = nvt
        ri[sl, :] = nit

    @pl.when(m_step == nm - 1)
    def _emit():
        vals_out[...] = rv[...][:, 0:_K]
        idx_out[...] = ri[...][:, 0:_K].astype(jnp.int32)


@jax.jit
def _run(task_desc, task_in, task_out, model_desc, model_in, model_out):
    nm = pl.cdiv(_M, _BM)
    q_spec = pl.BlockSpec((_Q, _D), lambda m: (0, 0))
    m_spec = pl.BlockSpec((_BM, _D), lambda m: (m, 0))
    out_spec = pl.BlockSpec((_Q, _K), lambda m: (0, 0))
    return pl.pallas_call(
        _topk_kernel,
        grid=(nm,),
        in_specs=[q_spec, q_spec, q_spec, m_spec, m_spec, m_spec],
        out_specs=[out_spec, out_spec],
        out_shape=[
            jax.ShapeDtypeStruct((_Q, _K), jnp.float32),
            jax.ShapeDtypeStruct((_Q, _K), jnp.int32),
        ],
        scratch_shapes=[
            pltpu.VMEM((_Q, _D), jnp.bfloat16),
            pltpu.VMEM((_Q, 2 * _D), jnp.bfloat16),
            pltpu.VMEM((_Q, 128), jnp.float32),
            pltpu.VMEM((_Q, 128), jnp.float32),
        ],
    )(task_desc, task_in, task_out, model_desc, model_in, model_out)


def kernel(task_desc, task_in, task_out, model_desc, model_in, model_out, top_k):
    vals, idx = _run(task_desc, task_in, task_out,
                     model_desc, model_in, model_out)
    return vals, idx
